# baseline (speedup 1.0000x reference)
import jax
import jax.numpy as jnp
import numpy as np
from jax import lax
from jax.experimental import pallas as pl
from jax.experimental.pallas import tpu as pltpu

EMBED_DIM = 64
SIGMA_D = 0.2
SIGMA_A = 15.0
ANGLE_K = 3
FACTOR_A = 180.0 / (SIGMA_A * np.pi)
N_POINTS = 1024
ROW_BLOCK = 64
COL_BLOCK = 128
KNN_ROW_BLOCK = 64

_DIV_TERM = np.exp(
    np.arange(0, EMBED_DIM, 2, dtype=np.float32) * (-np.log(10000.0) / EMBED_DIM)
).astype(np.float32).reshape(1, EMBED_DIM // 2)


def _dist_tile(rows, pts_t):
    px, py, pz = pts_t[0:1, :], pts_t[1:2, :], pts_t[2:3, :]
    rx, ry, rz = rows[:, 0:1], rows[:, 1:2], rows[:, 2:3]
    xy = jnp.dot(rows, pts_t, preferred_element_type=jnp.float32)
    x2a = px * px + py * py + pz * pz
    x2r = rx * rx + ry * ry + rz * rz
    return jnp.sqrt(jnp.maximum((x2r - 2.0 * xy) + x2a, 0.0))


def _knn_body(pts_t_ref, rows_ref, o_ref):
    r, n = KNN_ROW_BLOCK, N_POINTS
    pts_t = pts_t_ref[...]
    rows = rows_ref[...]
    px, py, pz = pts_t[0:1, :], pts_t[1:2, :], pts_t[2:3, :]
    rx, ry, rz = rows[:, 0:1], rows[:, 1:2], rows[:, 2:3]
    dist = _dist_tile(rows, pts_t)
    iota = lax.broadcasted_iota(jnp.int32, (r, n), 1)
    work = dist
    for t in range(ANGLE_K + 1):
        mn = jnp.min(work, axis=1, keepdims=True)
        sel = jnp.min(jnp.where(work == mn, iota, n), axis=1, keepdims=True)
        if t > 0:
            onehot = iota == sel
            nx = jnp.sum(jnp.where(onehot, px, 0.0), axis=1, keepdims=True)
            ny = jnp.sum(jnp.where(onehot, py, 0.0), axis=1, keepdims=True)
            nz = jnp.sum(jnp.where(onehot, pz, 0.0), axis=1, keepdims=True)
            base = 3 * (t - 1)
            o_ref[:, base:base + 1] = nx - rx
            o_ref[:, base + 1:base + 2] = ny - ry
            o_ref[:, base + 2:base + 3] = nz - rz
        work = jnp.where(iota == sel, jnp.float32(3.0e38), work)


def _embed(field, div3, w_sin, w_cos):
    r, c = field.shape
    args = field[:, :, None] * div3
    s = jnp.sin(args).reshape(r * c, EMBED_DIM // 2)
    co = jnp.cos(args).reshape(r * c, EMBED_DIM // 2)
    o = (jnp.dot(s, w_sin, preferred_element_type=jnp.float32)
         + jnp.dot(co, w_cos, preferred_element_type=jnp.float32))
    return o.reshape(r, c, EMBED_DIM)


def _emb_body(pts_c_ref, rows_ref, ev_ref, div_ref, wds_ref, wdc_ref,
              was_ref, wac_ref, bias_ref, out_ref):
    pts_c = pts_c_ref[...]
    rows = rows_ref[...]
    pcx, pcy, pcz = pts_c[0:1, :], pts_c[1:2, :], pts_c[2:3, :]
    rx, ry, rz = rows[:, 0:1], rows[:, 1:2], rows[:, 2:3]
    dist = _dist_tile(rows, pts_c)
    ax, ay, az = pcx - rx, pcy - ry, pcz - rz

    div3 = div_ref[...].reshape(1, 1, EMBED_DIM // 2)
    for k in range(ANGLE_K):
        ex = ev_ref[:, 3 * k:3 * k + 1]
        ey = ev_ref[:, 3 * k + 1:3 * k + 2]
        ez = ev_ref[:, 3 * k + 2:3 * k + 3]
        cx = ey * az - ez * ay
        cy = ez * ax - ex * az
        cz = ex * ay - ey * ax
        sinv = jnp.sqrt(cx * cx + cy * cy + cz * cz)
        cosv = ex * ax + ey * ay + ez * az
        cosv = jnp.where(cosv == 0.0, jnp.float32(0.0), cosv)
        ang = jnp.arctan2(sinv, cosv)
        e = _embed(ang * FACTOR_A, div3, was_ref[...], wac_ref[...])
        if k == 0:
            out_ref[...] = e
        else:
            out_ref[...] = jnp.maximum(out_ref[...], e)

    dout = _embed(dist / SIGMA_D, div3, wds_ref[...], wdc_ref[...])
    out_ref[...] = out_ref[...] + dout + bias_ref[...].reshape(1, 1, EMBED_DIM)


@jax.jit
def _run(pts, w_d, b_d, w_a, b_a):
    n = N_POINTS
    pts_t = pts.T

    evecs = pl.pallas_call(
        _knn_body,
        grid=(n // KNN_ROW_BLOCK,),
        in_specs=[
            pl.BlockSpec((3, n), lambda i: (0, 0)),
            pl.BlockSpec((KNN_ROW_BLOCK, 3), lambda i: (i, 0)),
        ],
        out_specs=pl.BlockSpec((KNN_ROW_BLOCK, 16), lambda i: (i, 0)),
        out_shape=jax.ShapeDtypeStruct((n, 16), jnp.float32),
        compiler_params=pltpu.CompilerParams(
            dimension_semantics=("arbitrary",),
        ),
    )(pts_t, pts)

    grid = (n // ROW_BLOCK, n // COL_BLOCK)
    out = pl.pallas_call(
        _emb_body,
        grid=grid,
        in_specs=[
            pl.BlockSpec((3, COL_BLOCK), lambda i, j: (0, j)),
            pl.BlockSpec((ROW_BLOCK, 3), lambda i, j: (i, 0)),
            pl.BlockSpec((ROW_BLOCK, 16), lambda i, j: (i, 0)),
            pl.BlockSpec((1, EMBED_DIM // 2), lambda i, j: (0, 0)),
            pl.BlockSpec((EMBED_DIM // 2, EMBED_DIM), lambda i, j: (0, 0)),
            pl.BlockSpec((EMBED_DIM // 2, EMBED_DIM), lambda i, j: (0, 0)),
            pl.BlockSpec((EMBED_DIM // 2, EMBED_DIM), lambda i, j: (0, 0)),
            pl.BlockSpec((EMBED_DIM // 2, EMBED_DIM), lambda i, j: (0, 0)),
            pl.BlockSpec((1, EMBED_DIM), lambda i, j: (0, 0)),
        ],
        out_specs=pl.BlockSpec((ROW_BLOCK, COL_BLOCK, EMBED_DIM),
                               lambda i, j: (i, j, 0)),
        out_shape=jax.ShapeDtypeStruct((n, n, EMBED_DIM), jnp.float32),
        compiler_params=pltpu.CompilerParams(
            dimension_semantics=("arbitrary", "arbitrary"),
        ),
    )(pts_t, pts, evecs, jnp.asarray(_DIV_TERM), w_d[:32], w_d[32:],
      w_a[:32], w_a[32:], (b_d + b_a).reshape(1, EMBED_DIM))
    return out[None]


def kernel(points, W_d, b_d, W_a, b_a):
    return _run(points[0], W_d, b_d, W_a, b_a)


# speedup vs baseline: 4.1194x; 4.1194x over previous
import jax
import jax.numpy as jnp
import numpy as np
from jax import lax
from jax.experimental import pallas as pl
from jax.experimental.pallas import tpu as pltpu

EMBED_DIM = 64
SIGMA_D = 0.2
SIGMA_A = 15.0
ANGLE_K = 3
FACTOR_A = 180.0 / (SIGMA_A * np.pi)
N_POINTS = 1024
ROW_BLOCK = 64
COL_BLOCK = 128
KNN_ROW_BLOCK = 64

_DIV_TERM = np.exp(
    np.arange(0, EMBED_DIM, 2, dtype=np.float32) * (-np.log(10000.0) / EMBED_DIM)
).astype(np.float32).reshape(1, EMBED_DIM // 2)


def _dist_tile(rows, pts_t):
    px, py, pz = pts_t[0:1, :], pts_t[1:2, :], pts_t[2:3, :]
    rx, ry, rz = rows[:, 0:1], rows[:, 1:2], rows[:, 2:3]
    xy = jnp.dot(rows, pts_t, preferred_element_type=jnp.float32)
    x2a = px * px + py * py + pz * pz
    x2r = rx * rx + ry * ry + rz * rz
    return jnp.sqrt(jnp.maximum((x2r - 2.0 * xy) + x2a, 0.0))


def _knn_body(pts_t_ref, rows_ref, o_ref):
    r, n = KNN_ROW_BLOCK, N_POINTS
    pts_t = pts_t_ref[...]
    rows = rows_ref[...]
    px, py, pz = pts_t[0:1, :], pts_t[1:2, :], pts_t[2:3, :]
    rx, ry, rz = rows[:, 0:1], rows[:, 1:2], rows[:, 2:3]
    dist = _dist_tile(rows, pts_t)
    iota = lax.broadcasted_iota(jnp.int32, (r, n), 1)
    work = dist
    for t in range(ANGLE_K + 1):
        mn = jnp.min(work, axis=1, keepdims=True)
        sel = jnp.min(jnp.where(work == mn, iota, n), axis=1, keepdims=True)
        if t > 0:
            onehot = iota == sel
            nx = jnp.sum(jnp.where(onehot, px, 0.0), axis=1, keepdims=True)
            ny = jnp.sum(jnp.where(onehot, py, 0.0), axis=1, keepdims=True)
            nz = jnp.sum(jnp.where(onehot, pz, 0.0), axis=1, keepdims=True)
            base = 3 * (t - 1)
            o_ref[:, base:base + 1] = nx - rx
            o_ref[:, base + 1:base + 2] = ny - ry
            o_ref[:, base + 2:base + 3] = nz - rz
        work = jnp.where(iota == sel, jnp.float32(3.0e38), work)


_INV_2PI = float(1.0 / (2.0 * np.pi))
_SC = (6.28305406, -41.33112111, 81.3654677, -74.47078941, 32.76852614)
_CC = (0.99999921, -19.73898031, 64.92865571, -85.27159959, 58.79037925,
       -21.07090497)


def _sincos(x):
    y = x * _INV_2PI
    r = y - jnp.floor(y + 0.5)
    u = r * r
    s = r * (_SC[0] + u * (_SC[1] + u * (_SC[2] + u * (_SC[3] + u * _SC[4]))))
    c = _CC[0] + u * (_CC[1] + u * (_CC[2] + u * (_CC[3] + u * (
        _CC[4] + u * _CC[5]))))
    return s, c


def _embed(field, div3, w_sin, w_cos):
    r, c = field.shape
    args = field[:, :, None] * div3
    s, co = _sincos(args)
    s = s.astype(jnp.bfloat16).reshape(r * c, EMBED_DIM // 2)
    co = co.astype(jnp.bfloat16).reshape(r * c, EMBED_DIM // 2)
    o = (jnp.dot(s, w_sin, preferred_element_type=jnp.float32)
         + jnp.dot(co, w_cos, preferred_element_type=jnp.float32))
    return o.reshape(r, c, EMBED_DIM)


def _emb_body(pts_c_ref, rows_ref, ev_ref, div_ref, wds_ref, wdc_ref,
              was_ref, wac_ref, bias_ref, out_ref):
    pts_c = pts_c_ref[...]
    rows = rows_ref[...]
    pcx, pcy, pcz = pts_c[0:1, :], pts_c[1:2, :], pts_c[2:3, :]
    rx, ry, rz = rows[:, 0:1], rows[:, 1:2], rows[:, 2:3]
    dist = _dist_tile(rows, pts_c)
    ax, ay, az = pcx - rx, pcy - ry, pcz - rz

    div3 = div_ref[...].reshape(1, 1, EMBED_DIM // 2)
    for k in range(ANGLE_K):
        ex = ev_ref[:, 3 * k:3 * k + 1]
        ey = ev_ref[:, 3 * k + 1:3 * k + 2]
        ez = ev_ref[:, 3 * k + 2:3 * k + 3]
        cx = ey * az - ez * ay
        cy = ez * ax - ex * az
        cz = ex * ay - ey * ax
        sinv = jnp.sqrt(cx * cx + cy * cy + cz * cz)
        cosv = ex * ax + ey * ay + ez * az
        cosv = jnp.where(cosv == 0.0, jnp.float32(0.0), cosv)
        ang = jnp.arctan2(sinv, cosv)
        e = _embed(ang * FACTOR_A, div3, was_ref[...], wac_ref[...])
        if k == 0:
            out_ref[...] = e
        else:
            out_ref[...] = jnp.maximum(out_ref[...], e)

    dout = _embed(dist / SIGMA_D, div3, wds_ref[...], wdc_ref[...])
    out_ref[...] = out_ref[...] + dout + bias_ref[...].reshape(1, 1, EMBED_DIM)


@jax.jit
def _run(pts, w_d, b_d, w_a, b_a):
    n = N_POINTS
    pts_t = pts.T

    evecs = pl.pallas_call(
        _knn_body,
        grid=(n // KNN_ROW_BLOCK,),
        in_specs=[
            pl.BlockSpec((3, n), lambda i: (0, 0)),
            pl.BlockSpec((KNN_ROW_BLOCK, 3), lambda i: (i, 0)),
        ],
        out_specs=pl.BlockSpec((KNN_ROW_BLOCK, 16), lambda i: (i, 0)),
        out_shape=jax.ShapeDtypeStruct((n, 16), jnp.float32),
        compiler_params=pltpu.CompilerParams(
            dimension_semantics=("arbitrary",),
        ),
    )(pts_t, pts)

    grid = (n // ROW_BLOCK, n // COL_BLOCK)
    out = pl.pallas_call(
        _emb_body,
        grid=grid,
        in_specs=[
            pl.BlockSpec((3, COL_BLOCK), lambda i, j: (0, j)),
            pl.BlockSpec((ROW_BLOCK, 3), lambda i, j: (i, 0)),
            pl.BlockSpec((ROW_BLOCK, 16), lambda i, j: (i, 0)),
            pl.BlockSpec((1, EMBED_DIM // 2), lambda i, j: (0, 0)),
            pl.BlockSpec((EMBED_DIM // 2, EMBED_DIM), lambda i, j: (0, 0)),
            pl.BlockSpec((EMBED_DIM // 2, EMBED_DIM), lambda i, j: (0, 0)),
            pl.BlockSpec((EMBED_DIM // 2, EMBED_DIM), lambda i, j: (0, 0)),
            pl.BlockSpec((EMBED_DIM // 2, EMBED_DIM), lambda i, j: (0, 0)),
            pl.BlockSpec((1, EMBED_DIM), lambda i, j: (0, 0)),
        ],
        out_specs=pl.BlockSpec((ROW_BLOCK, COL_BLOCK, EMBED_DIM),
                               lambda i, j: (i, j, 0)),
        out_shape=jax.ShapeDtypeStruct((n, n, EMBED_DIM), jnp.float32),
        compiler_params=pltpu.CompilerParams(
            dimension_semantics=("arbitrary", "arbitrary"),
        ),
    )(pts_t, pts, evecs, jnp.asarray(_DIV_TERM),
      w_d[:32].astype(jnp.bfloat16), w_d[32:].astype(jnp.bfloat16),
      w_a[:32].astype(jnp.bfloat16), w_a[32:].astype(jnp.bfloat16),
      (b_d + b_a).reshape(1, EMBED_DIM))
    return out[None]


def kernel(points, W_d, b_d, W_a, b_a):
    return _run(points[0], W_d, b_d, W_a, b_a)
